# baseline (speedup 1.0000x reference)
import functools

import jax
import jax.numpy as jnp
from jax import lax
from jax.experimental import pallas as pl
from jax.experimental.pallas import tpu as pltpu
from jax.experimental.pallas import tpu_sc as plsc

N = 10000
E = 320000
D = 128
G = 8
NP = 10240
CH = 128
NTILES = 32
CHUNKS = 80
EPT = CHUNKS * CH
EP = EPT * NTILES
RPT = NP // 16
BR = 1280
GRID = NP // BR

_mesh = plsc.VectorSubcoreMesh(core_axis_name="c", subcore_axis_name="s")


@functools.partial(
    pl.kernel, mesh=_mesh,
    out_type=jax.ShapeDtypeStruct((2, NP, D), jnp.float32),
    scratch_types=[
        pltpu.VMEM_SHARED((NP, D), jnp.float32),
        pltpu.VMEM((CHUNKS, CH), jnp.int32),
        pltpu.VMEM((CH, D), jnp.float32),
        pltpu.SemaphoreType.DMA,
        pltpu.SemaphoreType.DMA,
        pltpu.SemaphoreType.DMA,
        pltpu.SemaphoreType.DMA,
    ],
)
def _sc_degree(dst_hbm, deg_hbm, accd, didx, ones_v, ss0, ss1, ss2, ss3):
    cid = lax.axis_index("c")
    sid = lax.axis_index("s")
    ss = (ss0, ss1, ss2, ss3)

    def fill0(i, carry):
        for j in range(D // 16):
            ones_v[i, pl.ds(j * 16, 16)] = jnp.zeros((16,), jnp.float32)
        return carry
    lax.fori_loop(0, CH, fill0, 0)
    for j in range(RPT // CH):
        pltpu.sync_copy(ones_v, accd.at[pl.ds(sid * RPT + j * CH, CH)])

    def fill1(i, carry):
        for j in range(D // 16):
            ones_v[i, pl.ds(j * 16, 16)] = jnp.ones((16,), jnp.float32)
        return carry
    lax.fori_loop(0, CH, fill1, 0)

    tb = (cid * 16 + sid) * CHUNKS
    pltpu.sync_copy(dst_hbm.at[pl.ds(tb, CHUNKS)], didx)
    plsc.subcore_barrier()

    for b in range(3):
        pltpu.async_copy(ones_v, accd.at[didx.at[b]], ss[b], add=True)

    def quad(g, carry):
        for b in range(4):
            idx = g * 4 + b

            @pl.when(idx + 3 < CHUNKS)
            def _():
                pltpu.async_copy(ones_v, accd.at[didx.at[idx + 3]],
                                 ss[(b + 3) % 4], add=True)
            pltpu.make_async_copy(ones_v, accd.at[didx.at[idx]],
                                  ss[b]).wait()
        return carry
    lax.fori_loop(0, CHUNKS // 4, quad, 0)
    plsc.subcore_barrier()

    pltpu.sync_copy(accd.at[pl.ds(sid * RPT, RPT)],
                    deg_hbm.at[cid, pl.ds(sid * RPT, RPT)])


HC = 32
C0 = 160
C1 = CHUNKS * 2 - C0


@functools.partial(
    pl.kernel, mesh=_mesh,
    out_type=jax.ShapeDtypeStruct((2, NP, D), jnp.float32),
    scratch_types=[
        pltpu.VMEM_SHARED((NP, D), jnp.float32),
        pltpu.VMEM((HC, CH), jnp.int32),
        pltpu.VMEM((HC, CH), jnp.int32),
        pltpu.VMEM((CH, D), jnp.float32),
        pltpu.VMEM((CH, D), jnp.float32),
        pltpu.SemaphoreType.DMA,
        pltpu.SemaphoreType.DMA,
        pltpu.SemaphoreType.DMA,
        pltpu.SemaphoreType.DMA,
    ],
)
def _sc_spmm(t_hbm, src_hbm, dst_hbm, y_hbm, acc, sidx, didx,
             rows0, rows1, sg0, sg1, ss0, ss1):
    cid = lax.axis_index("c")
    sid = lax.axis_index("s")
    rows = (rows0, rows1)
    sg = (sg0, sg1)
    ss = (ss0, ss1)

    def fill(i, carry):
        for j in range(D // 16):
            rows0[i, pl.ds(j * 16, 16)] = jnp.zeros((16,), jnp.float32)
        return carry
    lax.fori_loop(0, CH, fill, 0)
    for j in range(RPT // CH):
        pltpu.sync_copy(rows0, acc.at[pl.ds(sid * RPT + j * CH, CH)])
    plsc.subcore_barrier()

    def run_stretch(tb, carry):
        pltpu.sync_copy(src_hbm.at[pl.ds(tb, HC)], sidx)
        pltpu.sync_copy(dst_hbm.at[pl.ds(tb, HC)], didx)
        pltpu.async_copy(t_hbm.at[sidx.at[0]], rows0, sg0)

        def pair(g, carry):
            for b in (0, 1):
                idx = g * 2 + b
                pltpu.make_async_copy(t_hbm.at[sidx.at[idx]], rows[b],
                                      sg[b]).wait()

                @pl.when(jnp.logical_and(idx + 1 < HC, idx > 0))
                def _():
                    pltpu.make_async_copy(rows[1 - b],
                                          acc.at[didx.at[idx]],
                                          ss[1 - b]).wait()

                @pl.when(idx + 1 < HC)
                def _():
                    pltpu.async_copy(t_hbm.at[sidx.at[idx + 1]],
                                     rows[1 - b], sg[1 - b])

                pltpu.async_copy(rows[b], acc.at[didx.at[idx]], ss[b],
                                 add=True)
            return carry
        lax.fori_loop(0, HC // 2, pair, 0)
        pltpu.make_async_copy(rows[0], acc.at[didx.at[0]], ss[0]).wait()
        pltpu.make_async_copy(rows[1], acc.at[didx.at[0]], ss[1]).wait()
        return carry

    @pl.when(cid == 0)
    def _():
        def body(h, carry):
            return run_stretch(sid * C0 + h * HC, carry)
        lax.fori_loop(0, C0 // HC, body, 0)

    if C1 > 0:
        @pl.when(cid == 1)
        def _():
            def body(h, carry):
                return run_stretch(16 * C0 + sid * C1 + h * HC, carry)
            lax.fori_loop(0, C1 // HC, body, 0)
    plsc.subcore_barrier()

    pltpu.sync_copy(acc.at[pl.ds(sid * RPT, RPT)],
                    y_hbm.at[cid, pl.ds(sid * RPT, RPT)])


def _prelu(o):
    return jnp.where(o >= 0.0, o, 0.25 * o)


def _tc_prep_body(degp_ref, x_ref, w_ref, dis_ref, t_ref, acc_ref):
    dp = degp_ref[...]
    deg = dp[0, :, :1] + dp[1, :, :1]
    dis = jnp.where(deg > 0.0, lax.rsqrt(jnp.maximum(deg, 1e-12)), 0.0)
    dis_b = jnp.broadcast_to(dis, (BR, D))
    x = x_ref[...]
    dis_ref[...] = dis_b
    t_ref[...] = dis_b * x
    acc_ref[...] = jnp.dot(x, w_ref[...], preferred_element_type=jnp.float32)


_tc_prep = pl.pallas_call(
    _tc_prep_body,
    grid=(GRID,),
    in_specs=[
        pl.BlockSpec((2, BR, D), lambda i: (0, i, 0)),
        pl.BlockSpec((BR, D), lambda i: (i, 0)),
        pl.BlockSpec((D, D), lambda i: (0, 0)),
    ],
    out_specs=[
        pl.BlockSpec((BR, D), lambda i: (i, 0)),
        pl.BlockSpec((BR, D), lambda i: (i, 0)),
        pl.BlockSpec((BR, D), lambda i: (i, 0)),
    ],
    out_shape=[
        jax.ShapeDtypeStruct((NP, D), jnp.float32),
        jax.ShapeDtypeStruct((NP, D), jnp.float32),
        jax.ShapeDtypeStruct((NP, D), jnp.float32),
    ],
)


def _tc_hop_body(y_ref, dis_ref, acc_ref, w_ref, t_ref, accout_ref):
    y = y_ref[...]
    dis = dis_ref[...]
    xk = dis * (y[0] + y[1])
    accout_ref[...] = acc_ref[...] + jnp.dot(
        xk, w_ref[...], preferred_element_type=jnp.float32)
    t_ref[...] = dis * xk


_tc_hop = pl.pallas_call(
    _tc_hop_body,
    grid=(GRID,),
    in_specs=[
        pl.BlockSpec((2, BR, D), lambda i: (0, i, 0)),
        pl.BlockSpec((BR, D), lambda i: (i, 0)),
        pl.BlockSpec((BR, D), lambda i: (i, 0)),
        pl.BlockSpec((D, D), lambda i: (0, 0)),
    ],
    out_specs=[
        pl.BlockSpec((BR, D), lambda i: (i, 0)),
        pl.BlockSpec((BR, D), lambda i: (i, 0)),
    ],
    out_shape=[
        jax.ShapeDtypeStruct((NP, D), jnp.float32),
        jax.ShapeDtypeStruct((NP, D), jnp.float32),
    ],
)


def _tc_l1_end_body(y_ref, dis_ref, acc_ref, w_ref, b_ref, w20_ref,
                    t_ref, acc2_ref):
    y = y_ref[...]
    dis = dis_ref[...]
    xk = dis * (y[0] + y[1])
    o = acc_ref[...] + jnp.dot(
        xk, w_ref[...], preferred_element_type=jnp.float32) + b_ref[...]
    h = _prelu(o)
    t_ref[...] = dis * h
    acc2_ref[...] = jnp.dot(h, w20_ref[...], preferred_element_type=jnp.float32)


_tc_l1_end = pl.pallas_call(
    _tc_l1_end_body,
    grid=(GRID,),
    in_specs=[
        pl.BlockSpec((2, BR, D), lambda i: (0, i, 0)),
        pl.BlockSpec((BR, D), lambda i: (i, 0)),
        pl.BlockSpec((BR, D), lambda i: (i, 0)),
        pl.BlockSpec((D, D), lambda i: (0, 0)),
        pl.BlockSpec((1, D), lambda i: (0, 0)),
        pl.BlockSpec((D, D), lambda i: (0, 0)),
    ],
    out_specs=[
        pl.BlockSpec((BR, D), lambda i: (i, 0)),
        pl.BlockSpec((BR, D), lambda i: (i, 0)),
    ],
    out_shape=[
        jax.ShapeDtypeStruct((NP, D), jnp.float32),
        jax.ShapeDtypeStruct((NP, D), jnp.float32),
    ],
)


def _tc_l2_end_body(y_ref, dis_ref, acc_ref, w_ref, b_ref, batch_ref,
                    wout_ref, bout_ref, out_ref, pool_ref):
    i = pl.program_id(0)
    y = y_ref[...]
    xk = dis_ref[...] * (y[0] + y[1])
    o = acc_ref[...] + jnp.dot(
        xk, w_ref[...], preferred_element_type=jnp.float32) + b_ref[...]
    h2 = _prelu(o)
    b = batch_ref[0]
    gids = lax.broadcasted_iota(jnp.int32, (G, BR), 0)
    onehot = (gids == b).astype(jnp.float32)
    part = jnp.dot(onehot, h2, preferred_element_type=jnp.float32)

    @pl.when(i == 0)
    def _():
        pool_ref[...] = part

    @pl.when(i > 0)
    def _():
        pool_ref[...] = pool_ref[...] + part

    @pl.when(i == GRID - 1)
    def _():
        out_ref[...] = jnp.dot(
            pool_ref[...], wout_ref[...],
            preferred_element_type=jnp.float32) + bout_ref[...]


_tc_l2_end = pl.pallas_call(
    _tc_l2_end_body,
    grid=(GRID,),
    in_specs=[
        pl.BlockSpec((2, BR, D), lambda i: (0, i, 0)),
        pl.BlockSpec((BR, D), lambda i: (i, 0)),
        pl.BlockSpec((BR, D), lambda i: (i, 0)),
        pl.BlockSpec((D, D), lambda i: (0, 0)),
        pl.BlockSpec((1, D), lambda i: (0, 0)),
        pl.BlockSpec((1, 1, BR), lambda i: (i, 0, 0)),
        pl.BlockSpec((D, D), lambda i: (0, 0)),
        pl.BlockSpec((1, D), lambda i: (0, 0)),
    ],
    out_specs=pl.BlockSpec((G, D), lambda i: (0, 0)),
    out_shape=jax.ShapeDtypeStruct((G, D), jnp.float32),
    scratch_shapes=[pltpu.VMEM((G, D), jnp.float32)],
)


def kernel(x, edge_index, batch, W1, b1, W2, b2, Wout, bout):
    src = edge_index[0]
    dst = edge_index[1]
    pad_e = EP - E
    srcp = jnp.concatenate([src, jnp.zeros((pad_e,), jnp.int32)])
    pad_dst = N + (jnp.arange(pad_e, dtype=jnp.int32) % (NP - N))
    dstp = jnp.concatenate([dst, pad_dst])
    src2 = srcp.reshape(NTILES * CHUNKS, CH)
    dst2 = dstp.reshape(NTILES * CHUNKS, CH)
    xp = jnp.pad(x, ((0, NP - N), (0, 0)))
    batchp = jnp.pad(batch, (0, NP - N), constant_values=G).reshape(GRID, 1, BR)
    b1r = b1.reshape(1, D)
    b2r = b2.reshape(1, D)
    woutp = jnp.pad(Wout, ((0, 0), (0, D - 1)))
    boutp = jnp.pad(bout, (0, D - 1)).reshape(1, D)

    degp = _sc_degree(dst2)
    dis_b, t, acc = _tc_prep(degp, xp, W1[0])
    for k in (1, 2):
        y = _sc_spmm(t, src2, dst2)
        t, acc = _tc_hop(y, dis_b, acc, W1[k])
    y = _sc_spmm(t, src2, dst2)
    t, acc = _tc_l1_end(y, dis_b, acc, W1[3], b1r, W2[0])
    for k in (1, 2):
        y = _sc_spmm(t, src2, dst2)
        t, acc = _tc_hop(y, dis_b, acc, W2[k])
    y = _sc_spmm(t, src2, dst2)
    out = _tc_l2_end(y, dis_b, acc, W2[3], b2r, batchp, woutp, boutp)
    return out[:, :1]


# speedup vs baseline: 1.0246x; 1.0246x over previous
import functools

import jax
import jax.numpy as jnp
from jax import lax
from jax.experimental import pallas as pl
from jax.experimental.pallas import tpu as pltpu
from jax.experimental.pallas import tpu_sc as plsc

N = 10000
E = 320000
D = 128
G = 8
NP = 10240
CH = 128
NTILES = 32
CHUNKS = 80
EPT = CHUNKS * CH
EP = EPT * NTILES
RPT = NP // 16
BR = 1280
GRID = NP // BR

_mesh = plsc.VectorSubcoreMesh(core_axis_name="c", subcore_axis_name="s")
_mesh1 = plsc.VectorSubcoreMesh(core_axis_name="c", subcore_axis_name="s",
                                num_cores=1)


@functools.partial(
    pl.kernel, mesh=_mesh,
    out_type=jax.ShapeDtypeStruct((2, NP, D), jnp.float32),
    scratch_types=[
        pltpu.VMEM_SHARED((NP, D), jnp.float32),
        pltpu.VMEM((CHUNKS, CH), jnp.int32),
        pltpu.VMEM((CH, D), jnp.float32),
        pltpu.SemaphoreType.DMA,
        pltpu.SemaphoreType.DMA,
        pltpu.SemaphoreType.DMA,
        pltpu.SemaphoreType.DMA,
    ],
)
def _sc_degree(dst_hbm, deg_hbm, accd, didx, ones_v, ss0, ss1, ss2, ss3):
    cid = lax.axis_index("c")
    sid = lax.axis_index("s")
    ss = (ss0, ss1, ss2, ss3)

    def fill0(i, carry):
        for j in range(D // 16):
            ones_v[i, pl.ds(j * 16, 16)] = jnp.zeros((16,), jnp.float32)
        return carry
    lax.fori_loop(0, CH, fill0, 0)
    for j in range(RPT // CH):
        pltpu.sync_copy(ones_v, accd.at[pl.ds(sid * RPT + j * CH, CH)])

    def fill1(i, carry):
        for j in range(D // 16):
            ones_v[i, pl.ds(j * 16, 16)] = jnp.ones((16,), jnp.float32)
        return carry
    lax.fori_loop(0, CH, fill1, 0)

    tb = (cid * 16 + sid) * CHUNKS
    pltpu.sync_copy(dst_hbm.at[pl.ds(tb, CHUNKS)], didx)
    plsc.subcore_barrier()

    for b in range(3):
        pltpu.async_copy(ones_v, accd.at[didx.at[b]], ss[b], add=True)

    def quad(g, carry):
        for b in range(4):
            idx = g * 4 + b

            @pl.when(idx + 3 < CHUNKS)
            def _():
                pltpu.async_copy(ones_v, accd.at[didx.at[idx + 3]],
                                 ss[(b + 3) % 4], add=True)
            pltpu.make_async_copy(ones_v, accd.at[didx.at[idx]],
                                  ss[b]).wait()
        return carry
    lax.fori_loop(0, CHUNKS // 4, quad, 0)
    plsc.subcore_barrier()

    pltpu.sync_copy(accd.at[pl.ds(sid * RPT, RPT)],
                    deg_hbm.at[cid, pl.ds(sid * RPT, RPT)])


HC = 32
C0 = 160


@functools.partial(
    pl.kernel, mesh=_mesh1,
    out_type=jax.ShapeDtypeStruct((NP, D), jnp.float32),
    scratch_types=[
        pltpu.VMEM_SHARED((NP, D), jnp.float32),
        pltpu.VMEM((HC, CH), jnp.int32),
        pltpu.VMEM((HC, CH), jnp.int32),
        pltpu.VMEM((CH, D), jnp.float32),
        pltpu.VMEM((CH, D), jnp.float32),
        pltpu.SemaphoreType.DMA,
        pltpu.SemaphoreType.DMA,
        pltpu.SemaphoreType.DMA,
        pltpu.SemaphoreType.DMA,
    ],
)
def _sc_spmm(t_hbm, src_hbm, dst_hbm, y_hbm, acc, sidx, didx,
             rows0, rows1, sg0, sg1, ss0, ss1):
    sid = lax.axis_index("s")
    rows = (rows0, rows1)
    sg = (sg0, sg1)
    ss = (ss0, ss1)

    def fill(i, carry):
        for j in range(D // 16):
            rows0[i, pl.ds(j * 16, 16)] = jnp.zeros((16,), jnp.float32)
        return carry
    lax.fori_loop(0, CH, fill, 0)
    for j in range(RPT // CH):
        pltpu.sync_copy(rows0, acc.at[pl.ds(sid * RPT + j * CH, CH)])
    plsc.subcore_barrier()

    def run_stretch(tb, carry):
        pltpu.sync_copy(src_hbm.at[pl.ds(tb, HC)], sidx)
        pltpu.sync_copy(dst_hbm.at[pl.ds(tb, HC)], didx)
        pltpu.async_copy(t_hbm.at[sidx.at[0]], rows0, sg0)

        def pair(g, carry):
            for b in (0, 1):
                idx = g * 2 + b
                pltpu.make_async_copy(t_hbm.at[sidx.at[idx]], rows[b],
                                      sg[b]).wait()

                @pl.when(jnp.logical_and(idx + 1 < HC, idx > 0))
                def _():
                    pltpu.make_async_copy(rows[1 - b],
                                          acc.at[didx.at[idx]],
                                          ss[1 - b]).wait()

                @pl.when(idx + 1 < HC)
                def _():
                    pltpu.async_copy(t_hbm.at[sidx.at[idx + 1]],
                                     rows[1 - b], sg[1 - b])

                pltpu.async_copy(rows[b], acc.at[didx.at[idx]], ss[b],
                                 add=True)
            return carry
        lax.fori_loop(0, HC // 2, pair, 0)
        pltpu.make_async_copy(rows[0], acc.at[didx.at[0]], ss[0]).wait()
        pltpu.make_async_copy(rows[1], acc.at[didx.at[0]], ss[1]).wait()
        return carry

    def body(h, carry):
        return run_stretch(sid * C0 + h * HC, carry)
    lax.fori_loop(0, C0 // HC, body, 0)
    plsc.subcore_barrier()

    pltpu.sync_copy(acc.at[pl.ds(sid * RPT, RPT)],
                    y_hbm.at[pl.ds(sid * RPT, RPT)])


def _prelu(o):
    return jnp.where(o >= 0.0, o, 0.25 * o)


def _tc_prep_body(degp_ref, x_ref, w_ref, dis_ref, t_ref, acc_ref):
    dp = degp_ref[...]
    deg = dp[0, :, :1] + dp[1, :, :1]
    dis = jnp.where(deg > 0.0, lax.rsqrt(jnp.maximum(deg, 1e-12)), 0.0)
    dis_b = jnp.broadcast_to(dis, (BR, D))
    x = x_ref[...]
    dis_ref[...] = dis_b
    t_ref[...] = dis_b * x
    acc_ref[...] = jnp.dot(x, w_ref[...], preferred_element_type=jnp.float32)


_tc_prep = pl.pallas_call(
    _tc_prep_body,
    grid=(GRID,),
    in_specs=[
        pl.BlockSpec((2, BR, D), lambda i: (0, i, 0)),
        pl.BlockSpec((BR, D), lambda i: (i, 0)),
        pl.BlockSpec((D, D), lambda i: (0, 0)),
    ],
    out_specs=[
        pl.BlockSpec((BR, D), lambda i: (i, 0)),
        pl.BlockSpec((BR, D), lambda i: (i, 0)),
        pl.BlockSpec((BR, D), lambda i: (i, 0)),
    ],
    out_shape=[
        jax.ShapeDtypeStruct((NP, D), jnp.float32),
        jax.ShapeDtypeStruct((NP, D), jnp.float32),
        jax.ShapeDtypeStruct((NP, D), jnp.float32),
    ],
)


def _tc_hop_body(y_ref, dis_ref, acc_ref, w_ref, t_ref, accout_ref):
    dis = dis_ref[...]
    xk = dis * y_ref[...]
    accout_ref[...] = acc_ref[...] + jnp.dot(
        xk, w_ref[...], preferred_element_type=jnp.float32)
    t_ref[...] = dis * xk


_tc_hop = pl.pallas_call(
    _tc_hop_body,
    grid=(GRID,),
    in_specs=[
        pl.BlockSpec((BR, D), lambda i: (i, 0)),
        pl.BlockSpec((BR, D), lambda i: (i, 0)),
        pl.BlockSpec((BR, D), lambda i: (i, 0)),
        pl.BlockSpec((D, D), lambda i: (0, 0)),
    ],
    out_specs=[
        pl.BlockSpec((BR, D), lambda i: (i, 0)),
        pl.BlockSpec((BR, D), lambda i: (i, 0)),
    ],
    out_shape=[
        jax.ShapeDtypeStruct((NP, D), jnp.float32),
        jax.ShapeDtypeStruct((NP, D), jnp.float32),
    ],
)


def _tc_l1_end_body(y_ref, dis_ref, acc_ref, w_ref, b_ref, w20_ref,
                    t_ref, acc2_ref):
    dis = dis_ref[...]
    xk = dis * y_ref[...]
    o = acc_ref[...] + jnp.dot(
        xk, w_ref[...], preferred_element_type=jnp.float32) + b_ref[...]
    h = _prelu(o)
    t_ref[...] = dis * h
    acc2_ref[...] = jnp.dot(h, w20_ref[...], preferred_element_type=jnp.float32)


_tc_l1_end = pl.pallas_call(
    _tc_l1_end_body,
    grid=(GRID,),
    in_specs=[
        pl.BlockSpec((BR, D), lambda i: (i, 0)),
        pl.BlockSpec((BR, D), lambda i: (i, 0)),
        pl.BlockSpec((BR, D), lambda i: (i, 0)),
        pl.BlockSpec((D, D), lambda i: (0, 0)),
        pl.BlockSpec((1, D), lambda i: (0, 0)),
        pl.BlockSpec((D, D), lambda i: (0, 0)),
    ],
    out_specs=[
        pl.BlockSpec((BR, D), lambda i: (i, 0)),
        pl.BlockSpec((BR, D), lambda i: (i, 0)),
    ],
    out_shape=[
        jax.ShapeDtypeStruct((NP, D), jnp.float32),
        jax.ShapeDtypeStruct((NP, D), jnp.float32),
    ],
)


def _tc_l2_end_body(y_ref, dis_ref, acc_ref, w_ref, b_ref, batch_ref,
                    wout_ref, bout_ref, out_ref, pool_ref):
    i = pl.program_id(0)
    xk = dis_ref[...] * y_ref[...]
    o = acc_ref[...] + jnp.dot(
        xk, w_ref[...], preferred_element_type=jnp.float32) + b_ref[...]
    h2 = _prelu(o)
    b = batch_ref[0]
    gids = lax.broadcasted_iota(jnp.int32, (G, BR), 0)
    onehot = (gids == b).astype(jnp.float32)
    part = jnp.dot(onehot, h2, preferred_element_type=jnp.float32)

    @pl.when(i == 0)
    def _():
        pool_ref[...] = part

    @pl.when(i > 0)
    def _():
        pool_ref[...] = pool_ref[...] + part

    @pl.when(i == GRID - 1)
    def _():
        out_ref[...] = jnp.dot(
            pool_ref[...], wout_ref[...],
            preferred_element_type=jnp.float32) + bout_ref[...]


_tc_l2_end = pl.pallas_call(
    _tc_l2_end_body,
    grid=(GRID,),
    in_specs=[
        pl.BlockSpec((BR, D), lambda i: (i, 0)),
        pl.BlockSpec((BR, D), lambda i: (i, 0)),
        pl.BlockSpec((BR, D), lambda i: (i, 0)),
        pl.BlockSpec((D, D), lambda i: (0, 0)),
        pl.BlockSpec((1, D), lambda i: (0, 0)),
        pl.BlockSpec((1, 1, BR), lambda i: (i, 0, 0)),
        pl.BlockSpec((D, D), lambda i: (0, 0)),
        pl.BlockSpec((1, D), lambda i: (0, 0)),
    ],
    out_specs=pl.BlockSpec((G, D), lambda i: (0, 0)),
    out_shape=jax.ShapeDtypeStruct((G, D), jnp.float32),
    scratch_shapes=[pltpu.VMEM((G, D), jnp.float32)],
)


def kernel(x, edge_index, batch, W1, b1, W2, b2, Wout, bout):
    src = edge_index[0]
    dst = edge_index[1]
    pad_e = EP - E
    srcp = jnp.concatenate([src, jnp.zeros((pad_e,), jnp.int32)])
    pad_dst = N + (jnp.arange(pad_e, dtype=jnp.int32) % (NP - N))
    dstp = jnp.concatenate([dst, pad_dst])
    src2 = srcp.reshape(NTILES * CHUNKS, CH)
    dst2 = dstp.reshape(NTILES * CHUNKS, CH)
    xp = jnp.pad(x, ((0, NP - N), (0, 0)))
    batchp = jnp.pad(batch, (0, NP - N), constant_values=G).reshape(GRID, 1, BR)
    b1r = b1.reshape(1, D)
    b2r = b2.reshape(1, D)
    woutp = jnp.pad(Wout, ((0, 0), (0, D - 1)))
    boutp = jnp.pad(bout, (0, D - 1)).reshape(1, D)

    degp = _sc_degree(dst2)
    dis_b, t, acc = _tc_prep(degp, xp, W1[0])
    for k in (1, 2):
        y = _sc_spmm(t, src2, dst2)
        t, acc = _tc_hop(y, dis_b, acc, W1[k])
    y = _sc_spmm(t, src2, dst2)
    t, acc = _tc_l1_end(y, dis_b, acc, W1[3], b1r, W2[0])
    for k in (1, 2):
        y = _sc_spmm(t, src2, dst2)
        t, acc = _tc_hop(y, dis_b, acc, W2[k])
    y = _sc_spmm(t, src2, dst2)
    out = _tc_l2_end(y, dis_b, acc, W2[3], b2r, batchp, woutp, boutp)
    return out[:, :1]


# speedup vs baseline: 2.2737x; 2.2191x over previous
import functools

import jax
import jax.numpy as jnp
from jax import lax
from jax.experimental import pallas as pl
from jax.experimental.pallas import tpu as pltpu
from jax.experimental.pallas import tpu_sc as plsc

N = 10000
E = 320000
D = 128
G = 8
NP = 10240
CH = 128
NTILES = 32
CHUNKS = 80
EPT = CHUNKS * CH
EP = EPT * NTILES
RPT = NP // 16
BR = 1280
GRID = NP // BR

_mesh = plsc.VectorSubcoreMesh(core_axis_name="c", subcore_axis_name="s")
_mesh1 = plsc.VectorSubcoreMesh(core_axis_name="c", subcore_axis_name="s",
                                num_cores=1)


@functools.partial(
    pl.kernel, mesh=_mesh,
    out_type=jax.ShapeDtypeStruct((2, NP, D), jnp.float32),
    scratch_types=[
        pltpu.VMEM_SHARED((NP, D), jnp.float32),
        pltpu.VMEM((CHUNKS, CH), jnp.int32),
        pltpu.VMEM((CH, D), jnp.float32),
        pltpu.SemaphoreType.DMA,
        pltpu.SemaphoreType.DMA,
        pltpu.SemaphoreType.DMA,
        pltpu.SemaphoreType.DMA,
    ],
)
def _sc_degree(dst_hbm, deg_hbm, accd, didx, ones_v, ss0, ss1, ss2, ss3):
    cid = lax.axis_index("c")
    sid = lax.axis_index("s")
    ss = (ss0, ss1, ss2, ss3)

    def fill0(i, carry):
        for j in range(D // 16):
            ones_v[i, pl.ds(j * 16, 16)] = jnp.zeros((16,), jnp.float32)
        return carry
    lax.fori_loop(0, CH, fill0, 0)
    for j in range(RPT // CH):
        pltpu.sync_copy(ones_v, accd.at[pl.ds(sid * RPT + j * CH, CH)])

    def fill1(i, carry):
        for j in range(D // 16):
            ones_v[i, pl.ds(j * 16, 16)] = jnp.ones((16,), jnp.float32)
        return carry
    lax.fori_loop(0, CH, fill1, 0)

    tb = (cid * 16 + sid) * CHUNKS
    pltpu.sync_copy(dst_hbm.at[pl.ds(tb, CHUNKS)], didx)
    plsc.subcore_barrier()

    for b in range(3):
        pltpu.async_copy(ones_v, accd.at[didx.at[b]], ss[b], add=True)

    def quad(g, carry):
        for b in range(4):
            idx = g * 4 + b

            @pl.when(idx + 3 < CHUNKS)
            def _():
                pltpu.async_copy(ones_v, accd.at[didx.at[idx + 3]],
                                 ss[(b + 3) % 4], add=True)
            pltpu.make_async_copy(ones_v, accd.at[didx.at[idx]],
                                  ss[b]).wait()
        return carry
    lax.fori_loop(0, CHUNKS // 4, quad, 0)
    plsc.subcore_barrier()

    pltpu.sync_copy(accd.at[pl.ds(sid * RPT, RPT)],
                    deg_hbm.at[cid, pl.ds(sid * RPT, RPT)])


HC = 32
C0 = 160


@functools.partial(
    pl.kernel, mesh=_mesh1,
    out_type=jax.ShapeDtypeStruct((NP, D), jnp.float32),
    scratch_types=[
        pltpu.VMEM_SHARED((NP, D), jnp.float32),
        pltpu.VMEM((HC, CH), jnp.int32),
        pltpu.VMEM((HC, CH), jnp.int32),
        pltpu.VMEM((CH, D), jnp.float32),
        pltpu.VMEM((CH, D), jnp.float32),
        pltpu.SemaphoreType.DMA,
        pltpu.SemaphoreType.DMA,
        pltpu.SemaphoreType.DMA,
        pltpu.SemaphoreType.DMA,
    ],
)
def _sc_spmm(t_hbm, src_hbm, dst_hbm, y_hbm, acc, sidx, didx,
             rows0, rows1, sg0, sg1, ss0, ss1):
    sid = lax.axis_index("s")
    rows = (rows0, rows1)
    sg = (sg0, sg1)
    ss = (ss0, ss1)

    def fill(i, carry):
        for j in range(D // 16):
            rows0[i, pl.ds(j * 16, 16)] = jnp.zeros((16,), jnp.float32)
        return carry
    lax.fori_loop(0, CH, fill, 0)
    for j in range(RPT // CH):
        pltpu.sync_copy(rows0, acc.at[pl.ds(sid * RPT + j * CH, CH)])
    plsc.subcore_barrier()

    def run_stretch(tb, carry):
        pltpu.sync_copy(src_hbm.at[pl.ds(tb, HC)], sidx)
        pltpu.sync_copy(dst_hbm.at[pl.ds(tb, HC)], didx)
        pltpu.async_copy(t_hbm.at[sidx.at[0]], rows0, sg0)

        def pair(g, carry):
            for b in (0, 1):
                idx = g * 2 + b
                pltpu.make_async_copy(t_hbm.at[sidx.at[idx]], rows[b],
                                      sg[b]).wait()

                @pl.when(jnp.logical_and(idx + 1 < HC, idx > 0))
                def _():
                    pltpu.make_async_copy(rows[1 - b],
                                          acc.at[didx.at[idx]],
                                          ss[1 - b]).wait()

                @pl.when(idx + 1 < HC)
                def _():
                    pltpu.async_copy(t_hbm.at[sidx.at[idx + 1]],
                                     rows[1 - b], sg[1 - b])

                pltpu.async_copy(rows[b], acc.at[didx.at[idx]], ss[b],
                                 add=True)
            return carry
        lax.fori_loop(0, HC // 2, pair, 0)
        pltpu.make_async_copy(rows[0], acc.at[didx.at[0]], ss[0]).wait()
        pltpu.make_async_copy(rows[1], acc.at[didx.at[0]], ss[1]).wait()
        return carry

    def body(h, carry):
        return run_stretch(sid * C0 + h * HC, carry)
    lax.fori_loop(0, C0 // HC, body, 0)
    plsc.subcore_barrier()

    pltpu.sync_copy(acc.at[pl.ds(sid * RPT, RPT)],
                    y_hbm.at[pl.ds(sid * RPT, RPT)])


def _prelu(o):
    return jnp.where(o >= 0.0, o, 0.25 * o)


def _tc_prep_body(degp_ref, x_ref, w_ref, dis_ref, t_ref, acc_ref):
    dp = degp_ref[...]
    deg = dp[0, :, :1] + dp[1, :, :1]
    dis = jnp.where(deg > 0.0, lax.rsqrt(jnp.maximum(deg, 1e-12)), 0.0)
    dis_b = jnp.broadcast_to(dis, (BR, D))
    x = x_ref[...]
    dis_ref[...] = dis_b
    t_ref[...] = dis_b * x
    acc_ref[...] = jnp.dot(x, w_ref[...], preferred_element_type=jnp.float32)


_tc_prep = pl.pallas_call(
    _tc_prep_body,
    grid=(GRID,),
    in_specs=[
        pl.BlockSpec((2, BR, D), lambda i: (0, i, 0)),
        pl.BlockSpec((BR, D), lambda i: (i, 0)),
        pl.BlockSpec((D, D), lambda i: (0, 0)),
    ],
    out_specs=[
        pl.BlockSpec((BR, D), lambda i: (i, 0)),
        pl.BlockSpec((BR, D), lambda i: (i, 0)),
        pl.BlockSpec((BR, D), lambda i: (i, 0)),
    ],
    out_shape=[
        jax.ShapeDtypeStruct((NP, D), jnp.float32),
        jax.ShapeDtypeStruct((NP, D), jnp.float32),
        jax.ShapeDtypeStruct((NP, D), jnp.float32),
    ],
)


def _tc_hop_body(y_ref, dis_ref, acc_ref, w_ref, t_ref, accout_ref):
    dis = dis_ref[...]
    xk = dis * y_ref[...]
    accout_ref[...] = acc_ref[...] + jnp.dot(
        xk, w_ref[...], preferred_element_type=jnp.float32)
    t_ref[...] = dis * xk


_tc_hop = pl.pallas_call(
    _tc_hop_body,
    grid=(GRID,),
    in_specs=[
        pl.BlockSpec((BR, D), lambda i: (i, 0)),
        pl.BlockSpec((BR, D), lambda i: (i, 0)),
        pl.BlockSpec((BR, D), lambda i: (i, 0)),
        pl.BlockSpec((D, D), lambda i: (0, 0)),
    ],
    out_specs=[
        pl.BlockSpec((BR, D), lambda i: (i, 0)),
        pl.BlockSpec((BR, D), lambda i: (i, 0)),
    ],
    out_shape=[
        jax.ShapeDtypeStruct((NP, D), jnp.float32),
        jax.ShapeDtypeStruct((NP, D), jnp.float32),
    ],
)


def _tc_l1_end_body(y_ref, dis_ref, acc_ref, w_ref, b_ref, w20_ref,
                    t_ref, acc2_ref):
    dis = dis_ref[...]
    xk = dis * y_ref[...]
    o = acc_ref[...] + jnp.dot(
        xk, w_ref[...], preferred_element_type=jnp.float32) + b_ref[...]
    h = _prelu(o)
    t_ref[...] = dis * h
    acc2_ref[...] = jnp.dot(h, w20_ref[...], preferred_element_type=jnp.float32)


_tc_l1_end = pl.pallas_call(
    _tc_l1_end_body,
    grid=(GRID,),
    in_specs=[
        pl.BlockSpec((BR, D), lambda i: (i, 0)),
        pl.BlockSpec((BR, D), lambda i: (i, 0)),
        pl.BlockSpec((BR, D), lambda i: (i, 0)),
        pl.BlockSpec((D, D), lambda i: (0, 0)),
        pl.BlockSpec((1, D), lambda i: (0, 0)),
        pl.BlockSpec((D, D), lambda i: (0, 0)),
    ],
    out_specs=[
        pl.BlockSpec((BR, D), lambda i: (i, 0)),
        pl.BlockSpec((BR, D), lambda i: (i, 0)),
    ],
    out_shape=[
        jax.ShapeDtypeStruct((NP, D), jnp.float32),
        jax.ShapeDtypeStruct((NP, D), jnp.float32),
    ],
)


def _tc_l2_end_body(y_ref, dis_ref, acc_ref, w_ref, b_ref, batch_ref,
                    wout_ref, bout_ref, out_ref, pool_ref):
    i = pl.program_id(0)
    xk = dis_ref[...] * y_ref[...]
    o = acc_ref[...] + jnp.dot(
        xk, w_ref[...], preferred_element_type=jnp.float32) + b_ref[...]
    h2 = _prelu(o)
    b = batch_ref[0]
    gids = lax.broadcasted_iota(jnp.int32, (G, BR), 0)
    onehot = (gids == b).astype(jnp.float32)
    part = jnp.dot(onehot, h2, preferred_element_type=jnp.float32)

    @pl.when(i == 0)
    def _():
        pool_ref[...] = part

    @pl.when(i > 0)
    def _():
        pool_ref[...] = pool_ref[...] + part

    @pl.when(i == GRID - 1)
    def _():
        out_ref[...] = jnp.dot(
            pool_ref[...], wout_ref[...],
            preferred_element_type=jnp.float32) + bout_ref[...]


_tc_l2_end = pl.pallas_call(
    _tc_l2_end_body,
    grid=(GRID,),
    in_specs=[
        pl.BlockSpec((BR, D), lambda i: (i, 0)),
        pl.BlockSpec((BR, D), lambda i: (i, 0)),
        pl.BlockSpec((BR, D), lambda i: (i, 0)),
        pl.BlockSpec((D, D), lambda i: (0, 0)),
        pl.BlockSpec((1, D), lambda i: (0, 0)),
        pl.BlockSpec((1, 1, BR), lambda i: (i, 0, 0)),
        pl.BlockSpec((D, D), lambda i: (0, 0)),
        pl.BlockSpec((1, D), lambda i: (0, 0)),
    ],
    out_specs=pl.BlockSpec((G, D), lambda i: (0, 0)),
    out_shape=jax.ShapeDtypeStruct((G, D), jnp.float32),
    scratch_shapes=[pltpu.VMEM((G, D), jnp.float32)],
)


def kernel(x, edge_index, batch, W1, b1, W2, b2, Wout, bout):
    src = edge_index[0]
    dst = edge_index[1]
    pad_e = EP - E
    pad_src = jnp.arange(pad_e, dtype=jnp.int32) % N
    srcp = jnp.concatenate([src, pad_src])
    pad_dst = N + (jnp.arange(pad_e, dtype=jnp.int32) % (NP - N))
    dstp = jnp.concatenate([dst, pad_dst])
    src2 = srcp.reshape(NTILES * CHUNKS, CH)
    dst2 = dstp.reshape(NTILES * CHUNKS, CH)
    xp = jnp.pad(x, ((0, NP - N), (0, 0)))
    batchp = jnp.pad(batch, (0, NP - N), constant_values=G).reshape(GRID, 1, BR)
    b1r = b1.reshape(1, D)
    b2r = b2.reshape(1, D)
    woutp = jnp.pad(Wout, ((0, 0), (0, D - 1)))
    boutp = jnp.pad(bout, (0, D - 1)).reshape(1, D)

    degp = _sc_degree(dst2)
    dis_b, t, acc = _tc_prep(degp, xp, W1[0])
    for k in (1, 2):
        y = _sc_spmm(t, src2, dst2)
        t, acc = _tc_hop(y, dis_b, acc, W1[k])
    y = _sc_spmm(t, src2, dst2)
    t, acc = _tc_l1_end(y, dis_b, acc, W1[3], b1r, W2[0])
    for k in (1, 2):
        y = _sc_spmm(t, src2, dst2)
        t, acc = _tc_hop(y, dis_b, acc, W2[k])
    y = _sc_spmm(t, src2, dst2)
    out = _tc_l2_end(y, dis_b, acc, W2[3], b2r, batchp, woutp, boutp)
    return out[:, :1]


# speedup vs baseline: 3.8264x; 1.6829x over previous
import functools

import jax
import jax.numpy as jnp
from jax import lax
from jax.experimental import pallas as pl
from jax.experimental.pallas import tpu as pltpu
from jax.experimental.pallas import tpu_sc as plsc

N = 10000
E = 320000
D = 128
G = 8
NP = 10240
CH = 128
NTILES = 32
CHUNKS = 80
EPT = CHUNKS * CH
EP = EPT * NTILES
RPT = NP // 16
BR = 1280
GRID = NP // BR

_mesh = plsc.VectorSubcoreMesh(core_axis_name="c", subcore_axis_name="s")
_mesh1 = plsc.VectorSubcoreMesh(core_axis_name="c", subcore_axis_name="s",
                                num_cores=1)


@functools.partial(
    pl.kernel, mesh=_mesh,
    out_type=jax.ShapeDtypeStruct((2, NP, D), jnp.float32),
    scratch_types=[
        pltpu.VMEM_SHARED((NP, D), jnp.float32),
        pltpu.VMEM((CHUNKS, CH), jnp.int32),
        pltpu.VMEM((CH, D), jnp.float32),
        pltpu.SemaphoreType.DMA,
        pltpu.SemaphoreType.DMA,
        pltpu.SemaphoreType.DMA,
        pltpu.SemaphoreType.DMA,
    ],
)
def _sc_degree(dst_hbm, deg_hbm, accd, didx, ones_v, ss0, ss1, ss2, ss3):
    cid = lax.axis_index("c")
    sid = lax.axis_index("s")
    ss = (ss0, ss1, ss2, ss3)

    def fill0(i, carry):
        for j in range(D // 16):
            ones_v[i, pl.ds(j * 16, 16)] = jnp.zeros((16,), jnp.float32)
        return carry
    lax.fori_loop(0, CH, fill0, 0)
    for j in range(RPT // CH):
        pltpu.sync_copy(ones_v, accd.at[pl.ds(sid * RPT + j * CH, CH)])

    def fill1(i, carry):
        for j in range(D // 16):
            ones_v[i, pl.ds(j * 16, 16)] = jnp.ones((16,), jnp.float32)
        return carry
    lax.fori_loop(0, CH, fill1, 0)

    tb = (cid * 16 + sid) * CHUNKS
    pltpu.sync_copy(dst_hbm.at[pl.ds(tb, CHUNKS)], didx)
    plsc.subcore_barrier()

    for b in range(3):
        pltpu.async_copy(ones_v, accd.at[didx.at[b]], ss[b], add=True)

    def quad(g, carry):
        for b in range(4):
            idx = g * 4 + b

            @pl.when(idx + 3 < CHUNKS)
            def _():
                pltpu.async_copy(ones_v, accd.at[didx.at[idx + 3]],
                                 ss[(b + 3) % 4], add=True)
            pltpu.make_async_copy(ones_v, accd.at[didx.at[idx]],
                                  ss[b]).wait()
        return carry
    lax.fori_loop(0, CHUNKS // 4, quad, 0)
    plsc.subcore_barrier()

    pltpu.sync_copy(accd.at[pl.ds(sid * RPT, RPT)],
                    deg_hbm.at[cid, pl.ds(sid * RPT, RPT)])


HC = 40
C0 = 80


@functools.partial(
    pl.kernel, mesh=_mesh,
    out_type=jax.ShapeDtypeStruct((2, NP, D), jnp.float32),
    scratch_types=[
        pltpu.VMEM_SHARED((NP, D), jnp.float32),
        pltpu.VMEM((HC, CH), jnp.int32),
        pltpu.VMEM((HC, CH), jnp.int32),
        pltpu.VMEM((CH, D), jnp.float32),
        pltpu.VMEM((CH, D), jnp.float32),
        pltpu.SemaphoreType.DMA,
        pltpu.SemaphoreType.DMA,
        pltpu.SemaphoreType.DMA,
        pltpu.SemaphoreType.DMA,
    ],
)
def _sc_spmm(t_hbm, src_hbm, dst_hbm, y_hbm, acc, sidx, didx,
             rows0, rows1, sg0, sg1, ss0, ss1):
    cid = lax.axis_index("c")
    sid = lax.axis_index("s")
    rows = (rows0, rows1)
    sg = (sg0, sg1)
    ss = (ss0, ss1)

    def fill(i, carry):
        for j in range(D // 16):
            rows0[i, pl.ds(j * 16, 16)] = jnp.zeros((16,), jnp.float32)
        return carry
    lax.fori_loop(0, CH, fill, 0)
    for j in range(RPT // CH):
        pltpu.sync_copy(rows0, acc.at[pl.ds(sid * RPT + j * CH, CH)])
    plsc.subcore_barrier()

    def run_stretch(tb, carry):
        pltpu.sync_copy(src_hbm.at[pl.ds(tb, HC)], sidx)
        pltpu.sync_copy(dst_hbm.at[pl.ds(tb, HC)], didx)
        pltpu.async_copy(t_hbm.at[sidx.at[0]], rows0, sg0)

        def pair(g, carry):
            for b in (0, 1):
                idx = g * 2 + b
                pltpu.make_async_copy(t_hbm.at[sidx.at[idx]], rows[b],
                                      sg[b]).wait()

                @pl.when(jnp.logical_and(idx + 1 < HC, idx > 0))
                def _():
                    pltpu.make_async_copy(rows[1 - b],
                                          acc.at[didx.at[idx]],
                                          ss[1 - b]).wait()

                @pl.when(idx + 1 < HC)
                def _():
                    pltpu.async_copy(t_hbm.at[sidx.at[idx + 1]],
                                     rows[1 - b], sg[1 - b])

                pltpu.async_copy(rows[b], acc.at[didx.at[idx]], ss[b],
                                 add=True)
            return carry
        lax.fori_loop(0, HC // 2, pair, 0)
        pltpu.make_async_copy(rows[0], acc.at[didx.at[0]], ss[0]).wait()
        pltpu.make_async_copy(rows[1], acc.at[didx.at[0]], ss[1]).wait()
        return carry

    def body(h, carry):
        return run_stretch((cid * 16 + sid) * C0 + h * HC, carry)
    lax.fori_loop(0, C0 // HC, body, 0)
    plsc.subcore_barrier()

    pltpu.sync_copy(acc.at[pl.ds(sid * RPT, RPT)],
                    y_hbm.at[cid, pl.ds(sid * RPT, RPT)])


def _prelu(o):
    return jnp.where(o >= 0.0, o, 0.25 * o)


def _tc_prep_body(degp_ref, x_ref, w_ref, dis_ref, t_ref, acc_ref):
    dp = degp_ref[...]
    deg = dp[0, :, :1] + dp[1, :, :1]
    dis = jnp.where(deg > 0.0, lax.rsqrt(jnp.maximum(deg, 1e-12)), 0.0)
    dis_b = jnp.broadcast_to(dis, (BR, D))
    x = x_ref[...]
    dis_ref[...] = dis_b
    t_ref[...] = dis_b * x
    acc_ref[...] = jnp.dot(x, w_ref[...], preferred_element_type=jnp.float32)


_tc_prep = pl.pallas_call(
    _tc_prep_body,
    grid=(GRID,),
    in_specs=[
        pl.BlockSpec((2, BR, D), lambda i: (0, i, 0)),
        pl.BlockSpec((BR, D), lambda i: (i, 0)),
        pl.BlockSpec((D, D), lambda i: (0, 0)),
    ],
    out_specs=[
        pl.BlockSpec((BR, D), lambda i: (i, 0)),
        pl.BlockSpec((BR, D), lambda i: (i, 0)),
        pl.BlockSpec((BR, D), lambda i: (i, 0)),
    ],
    out_shape=[
        jax.ShapeDtypeStruct((NP, D), jnp.float32),
        jax.ShapeDtypeStruct((NP, D), jnp.float32),
        jax.ShapeDtypeStruct((NP, D), jnp.float32),
    ],
)


def _tc_hop_body(y_ref, dis_ref, acc_ref, w_ref, t_ref, accout_ref):
    dis = dis_ref[...]
    y = y_ref[...]
    xk = dis * (y[0] + y[1])
    accout_ref[...] = acc_ref[...] + jnp.dot(
        xk, w_ref[...], preferred_element_type=jnp.float32)
    t_ref[...] = dis * xk


_tc_hop = pl.pallas_call(
    _tc_hop_body,
    grid=(GRID,),
    in_specs=[
        pl.BlockSpec((2, BR, D), lambda i: (0, i, 0)),
        pl.BlockSpec((BR, D), lambda i: (i, 0)),
        pl.BlockSpec((BR, D), lambda i: (i, 0)),
        pl.BlockSpec((D, D), lambda i: (0, 0)),
    ],
    out_specs=[
        pl.BlockSpec((BR, D), lambda i: (i, 0)),
        pl.BlockSpec((BR, D), lambda i: (i, 0)),
    ],
    out_shape=[
        jax.ShapeDtypeStruct((NP, D), jnp.float32),
        jax.ShapeDtypeStruct((NP, D), jnp.float32),
    ],
)


def _tc_l1_end_body(y_ref, dis_ref, acc_ref, w_ref, b_ref, w20_ref,
                    t_ref, acc2_ref):
    dis = dis_ref[...]
    y = y_ref[...]
    xk = dis * (y[0] + y[1])
    o = acc_ref[...] + jnp.dot(
        xk, w_ref[...], preferred_element_type=jnp.float32) + b_ref[...]
    h = _prelu(o)
    t_ref[...] = dis * h
    acc2_ref[...] = jnp.dot(h, w20_ref[...], preferred_element_type=jnp.float32)


_tc_l1_end = pl.pallas_call(
    _tc_l1_end_body,
    grid=(GRID,),
    in_specs=[
        pl.BlockSpec((2, BR, D), lambda i: (0, i, 0)),
        pl.BlockSpec((BR, D), lambda i: (i, 0)),
        pl.BlockSpec((BR, D), lambda i: (i, 0)),
        pl.BlockSpec((D, D), lambda i: (0, 0)),
        pl.BlockSpec((1, D), lambda i: (0, 0)),
        pl.BlockSpec((D, D), lambda i: (0, 0)),
    ],
    out_specs=[
        pl.BlockSpec((BR, D), lambda i: (i, 0)),
        pl.BlockSpec((BR, D), lambda i: (i, 0)),
    ],
    out_shape=[
        jax.ShapeDtypeStruct((NP, D), jnp.float32),
        jax.ShapeDtypeStruct((NP, D), jnp.float32),
    ],
)


def _tc_l2_end_body(y_ref, dis_ref, acc_ref, w_ref, b_ref, batch_ref,
                    wout_ref, bout_ref, out_ref, pool_ref):
    i = pl.program_id(0)
    y = y_ref[...]
    xk = dis_ref[...] * (y[0] + y[1])
    o = acc_ref[...] + jnp.dot(
        xk, w_ref[...], preferred_element_type=jnp.float32) + b_ref[...]
    h2 = _prelu(o)
    b = batch_ref[0]
    gids = lax.broadcasted_iota(jnp.int32, (G, BR), 0)
    onehot = (gids == b).astype(jnp.float32)
    part = jnp.dot(onehot, h2, preferred_element_type=jnp.float32)

    @pl.when(i == 0)
    def _():
        pool_ref[...] = part

    @pl.when(i > 0)
    def _():
        pool_ref[...] = pool_ref[...] + part

    @pl.when(i == GRID - 1)
    def _():
        out_ref[...] = jnp.dot(
            pool_ref[...], wout_ref[...],
            preferred_element_type=jnp.float32) + bout_ref[...]


_tc_l2_end = pl.pallas_call(
    _tc_l2_end_body,
    grid=(GRID,),
    in_specs=[
        pl.BlockSpec((2, BR, D), lambda i: (0, i, 0)),
        pl.BlockSpec((BR, D), lambda i: (i, 0)),
        pl.BlockSpec((BR, D), lambda i: (i, 0)),
        pl.BlockSpec((D, D), lambda i: (0, 0)),
        pl.BlockSpec((1, D), lambda i: (0, 0)),
        pl.BlockSpec((1, 1, BR), lambda i: (i, 0, 0)),
        pl.BlockSpec((D, D), lambda i: (0, 0)),
        pl.BlockSpec((1, D), lambda i: (0, 0)),
    ],
    out_specs=pl.BlockSpec((G, D), lambda i: (0, 0)),
    out_shape=jax.ShapeDtypeStruct((G, D), jnp.float32),
    scratch_shapes=[pltpu.VMEM((G, D), jnp.float32)],
)


def kernel(x, edge_index, batch, W1, b1, W2, b2, Wout, bout):
    src = edge_index[0]
    dst = edge_index[1]
    pad_e = EP - E
    pad_src = jnp.arange(pad_e, dtype=jnp.int32) % N
    srcp = jnp.concatenate([src, pad_src])
    pad_dst = N + (jnp.arange(pad_e, dtype=jnp.int32) % (NP - N))
    dstp = jnp.concatenate([dst, pad_dst])
    src2 = srcp.reshape(NTILES * CHUNKS, CH)
    dst2 = dstp.reshape(NTILES * CHUNKS, CH)
    xp = jnp.pad(x, ((0, NP - N), (0, 0)))
    batchp = jnp.pad(batch, (0, NP - N), constant_values=G).reshape(GRID, 1, BR)
    b1r = b1.reshape(1, D)
    b2r = b2.reshape(1, D)
    woutp = jnp.pad(Wout, ((0, 0), (0, D - 1)))
    boutp = jnp.pad(bout, (0, D - 1)).reshape(1, D)

    degp = _sc_degree(dst2)
    dis_b, t, acc = _tc_prep(degp, xp, W1[0])
    for k in (1, 2):
        y = _sc_spmm(t, src2, dst2)
        t, acc = _tc_hop(y, dis_b, acc, W1[k])
    y = _sc_spmm(t, src2, dst2)
    t, acc = _tc_l1_end(y, dis_b, acc, W1[3], b1r, W2[0])
    for k in (1, 2):
        y = _sc_spmm(t, src2, dst2)
        t, acc = _tc_hop(y, dis_b, acc, W2[k])
    y = _sc_spmm(t, src2, dst2)
    out = _tc_l2_end(y, dis_b, acc, W2[3], b2r, batchp, woutp, boutp)
    return out[:, :1]
